# Initial kernel scaffold; baseline (speedup 1.0000x reference)
import jax
import jax.numpy as jnp
from jax.experimental import pallas as pl


def kernel(labels, table):
    raise NotImplementedError("write your pallas kernel here")


# speedup vs baseline: 2.2416x; 2.2416x over previous
import functools

import jax
import jax.numpy as jnp
from jax import lax
from jax.experimental import pallas as pl
from jax.experimental.pallas import tpu as pltpu
from jax.experimental.pallas import tpu_sc as plsc

_INFO = plsc.get_sparse_core_info()
_NC = _INFO.num_cores
_NS = _INFO.num_subcores
_NW = _NC * _NS
_CHUNK = 128


@functools.partial(jax.jit, static_argnames=())
def _embed_lookup(labels2d, table):
    n_rows, chunk = labels2d.shape
    v, d = table.shape
    b = n_rows * chunk
    b_per_w = b // _NW
    nch = b_per_w // chunk

    mesh = plsc.VectorSubcoreMesh(core_axis_name="c", subcore_axis_name="s")

    @functools.partial(
        pl.kernel,
        mesh=mesh,
        out_type=jax.ShapeDtypeStruct((b, d), jnp.float32),
        scratch_types=[
            pltpu.VMEM((nch, chunk), jnp.int32),
            pltpu.VMEM((b_per_w, d), jnp.float32),
            pltpu.SemaphoreType.DMA,
        ],
    )
    def run(labels_hbm, table_hbm, out_hbm, idx_v, rows_v, sem):
        wid = lax.axis_index("s") * _NC + lax.axis_index("c")
        row_base = wid * nch
        pltpu.sync_copy(labels_hbm.at[pl.ds(row_base, nch)], idx_v)
        copies = []
        for j in range(nch):
            copies.append(
                pltpu.async_copy(
                    table_hbm.at[idx_v.at[j]],
                    rows_v.at[pl.ds(j * chunk, chunk)],
                    sem,
                )
            )
        for c in copies:
            c.wait()
        pltpu.sync_copy(rows_v, out_hbm.at[pl.ds(wid * b_per_w, b_per_w)])

    return run(labels2d, table)


def kernel(labels, table):
    (b,) = labels.shape
    labels2d = labels.astype(jnp.int32).reshape(b // _CHUNK, _CHUNK)
    return _embed_lookup(labels2d, table)
